# baseline (speedup 1.0000x reference)
import dataclasses
import functools

import jax
import jax.numpy as jnp
from jax import lax
from jax.experimental import pallas as pl
from jax.experimental.pallas import tpu as pltpu
from jax.experimental.pallas import tpu_sc as plsc

_B, _D, _K, _C = 512, 256, 128, 1000
_M = 100000
_SIGMA = 10.0
_GC = 1.0 / (2.0 * _SIGMA ** 2)

_NC, _NS, _L = 2, 16, 16
_NW = _NC * _NS
_BPW = _B // _NW
_CP = 1024
_NS16 = _D // _L
_NG = _K // _L
_PS = 17


def _sc_body(feat_hbm, cent_hbm, lab_hbm, neigh_hbm, w_hbm, hist_hbm,
             idx_all, feat_all, rows_a, rows_b, w_a, w_b, lbl_a, lbl_b,
             pacc_v, hist_a, hist_b, sem_a, sem_b, sem_ha, sem_hb):
    wid = lax.axis_index("s") * _NC + lax.axis_index("c")
    base = wid * _BPW
    zero = jnp.zeros((_L,), jnp.float32)
    k17 = [(lax.iota(jnp.int32, _L) + _L * g) * _PS for g in range(_NG)]

    def issue(i, rows_v, w_v, lbl_v, sem):
        idx = idx_all.at[i]
        pltpu.make_async_copy(cent_hbm.at[idx], rows_v, sem).start()
        pltpu.make_async_copy(w_hbm.at[idx], w_v, sem).start()
        pltpu.make_async_copy(lab_hbm.at[idx], lbl_v, sem).start()

    def wait(i, rows_v, w_v, lbl_v, sem):
        idx = idx_all.at[i]
        pltpu.make_async_copy(cent_hbm.at[idx], rows_v, sem).wait()
        pltpu.make_async_copy(w_hbm.at[idx], w_v, sem).wait()
        pltpu.make_async_copy(lab_hbm.at[idx], lbl_v, sem).wait()

    def compute(i, j, rows_v, w_v, lbl_v, hist_v, sem_h):
        @pl.when(j > 0)
        def _():
            pltpu.make_async_copy(hist_v, hist_hbm.at[base + i - 2], sem_h
                                  ).wait()
        for t in range(_CP // _L):
            hist_v[pl.ds(_L * t, _L)] = zero

        fs = [feat_all[i, pl.ds(_L * s, _L)] for s in range(_NS16)]

        @plsc.parallel_loop(0, _K, unroll=2)
        def _per_neighbour(k):
            acc = [zero, zero, zero, zero]
            for s in range(_NS16):
                diff = fs[s] - rows_v[k, pl.ds(_L * s, _L)]
                acc[s % 4] = acc[s % 4] + diff * diff
            pacc_v[pl.ds(k * _PS, _L)] = (acc[0] + acc[1]) + (acc[2] + acc[3])

        for g in range(_NG):
            part = [None] * 4
            for s in range(_NS16):
                col = plsc.load_gather(pacc_v, [k17[g] + s])
                part[s % 4] = col if part[s % 4] is None else part[s % 4] + col
            dist = (part[0] + part[1]) + (part[2] + part[3])
            ev = jnp.exp(w_v[pl.ds(_L * g, _L)] - _GC * dist)
            plsc.addupdate_scatter(hist_v, [lbl_v[pl.ds(_L * g, _L)]], ev)

        pltpu.make_async_copy(hist_v, hist_hbm.at[base + i], sem_h).start()

    pltpu.sync_copy(neigh_hbm.at[pl.ds(base, _BPW)], idx_all)
    pltpu.sync_copy(feat_hbm.at[pl.ds(base, _BPW)], feat_all)
    issue(0, rows_a, w_a, lbl_a, sem_a)

    @pl.loop(0, _BPW // 2)
    def _pair(j):
        a = 2 * j
        issue(a + 1, rows_b, w_b, lbl_b, sem_b)
        wait(a, rows_a, w_a, lbl_a, sem_a)
        compute(a, j, rows_a, w_a, lbl_a, hist_a, sem_ha)

        @pl.when(j < _BPW // 2 - 1)
        def _():
            issue(a + 2, rows_a, w_a, lbl_a, sem_a)

        wait(a + 1, rows_b, w_b, lbl_b, sem_b)
        compute(a + 1, j, rows_b, w_b, lbl_b, hist_b, sem_hb)

    pltpu.make_async_copy(hist_a, hist_hbm.at[base + _BPW - 2], sem_ha).wait()
    pltpu.make_async_copy(hist_b, hist_hbm.at[base + _BPW - 1], sem_hb).wait()


def _sc_class_sums(features, centres, centre_labels, neighbours, weight):
    mesh = plsc.VectorSubcoreMesh(core_axis_name="c", subcore_axis_name="s")
    cp = pltpu.CompilerParams()
    if "needs_layout_passes" in pltpu.CompilerParams.__dataclass_fields__:
        cp = dataclasses.replace(cp, needs_layout_passes=False)
    run = functools.partial(
        pl.kernel,
        out_type=jax.ShapeDtypeStruct((_B, _CP), jnp.float32),
        mesh=mesh,
        compiler_params=cp,
        scratch_types=[
            pltpu.VMEM((_BPW, _K), jnp.int32),
            pltpu.VMEM((_BPW, _D), jnp.float32),
            pltpu.VMEM((_K, _D), jnp.float32),
            pltpu.VMEM((_K, _D), jnp.float32),
            pltpu.VMEM((_K,), jnp.float32),
            pltpu.VMEM((_K,), jnp.float32),
            pltpu.VMEM((_K,), jnp.int32),
            pltpu.VMEM((_K,), jnp.int32),
            pltpu.VMEM((_K * _PS,), jnp.float32),
            pltpu.VMEM((_CP,), jnp.float32),
            pltpu.VMEM((_CP,), jnp.float32),
            pltpu.SemaphoreType.DMA,
            pltpu.SemaphoreType.DMA,
            pltpu.SemaphoreType.DMA,
            pltpu.SemaphoreType.DMA,
        ],
    )(_sc_body)
    return run(features, centres, centre_labels, neighbours, weight)


def _finalize_body(h_ref, o_ref):
    p = h_ref[...]
    p = jnp.where(p == 0.0, 1e-10, p)
    real = jax.lax.broadcasted_iota(jnp.int32, p.shape, 1) < _C
    s = jnp.sum(jnp.where(real, p, 0.0), axis=1, keepdims=True)
    logp = jnp.log(p / s)
    o_ref[...] = logp[:, :_C]


_finalize = pl.pallas_call(
    _finalize_body,
    grid=(1,),
    in_specs=[pl.BlockSpec((_B, _CP), lambda i: (0, 0))],
    out_specs=pl.BlockSpec((_B, _C), lambda i: (0, 0)),
    out_shape=jax.ShapeDtypeStruct((_B, _C), jnp.float32),
)


def kernel(features, centres, centre_labels, neighbours, weight):
    hist = _sc_class_sums(
        features,
        centres,
        centre_labels.astype(jnp.int32),
        neighbours.astype(jnp.int32),
        weight,
    )
    return _finalize(hist)


# speedup vs baseline: 1.0105x; 1.0105x over previous
import dataclasses
import functools

import jax
import jax.numpy as jnp
from jax import lax
from jax.experimental import pallas as pl
from jax.experimental.pallas import tpu as pltpu
from jax.experimental.pallas import tpu_sc as plsc

_B, _D, _K, _C = 512, 256, 128, 1000
_M = 100000
_SIGMA = 10.0
_GC = 1.0 / (2.0 * _SIGMA ** 2)

_NC, _NS, _L = 2, 16, 16
_NW = _NC * _NS
_BPW = _B // _NW
_CP = 1024
_NS16 = _D // _L
_NG = _K // _L
_PS = 17


def _sc_body(feat_hbm, cent_hbm, lab_hbm, neigh_hbm, w_hbm, hist_hbm,
             idx_all, feat_all, rows_a, rows_b, w_a, w_b, lbl_a, lbl_b,
             pacc_v, hist_a, hist_b, sem_a, sem_b, sem_ha, sem_hb):
    wid = lax.axis_index("s") * _NC + lax.axis_index("c")
    base = wid * _BPW
    zero = jnp.zeros((_L,), jnp.float32)
    k17 = [(lax.iota(jnp.int32, _L) + _L * g) * _PS for g in range(_NG)]

    def issue(i, rows_v, w_v, lbl_v, sem):
        idx = idx_all.at[i]
        pltpu.make_async_copy(cent_hbm.at[idx], rows_v, sem).start()
        pltpu.make_async_copy(w_hbm.at[idx], w_v, sem).start()
        pltpu.make_async_copy(lab_hbm.at[idx], lbl_v, sem).start()

    def wait(i, rows_v, w_v, lbl_v, sem):
        idx = idx_all.at[i]
        pltpu.make_async_copy(cent_hbm.at[idx], rows_v, sem).wait()
        pltpu.make_async_copy(w_hbm.at[idx], w_v, sem).wait()
        pltpu.make_async_copy(lab_hbm.at[idx], lbl_v, sem).wait()

    def compute(i, j, rows_v, w_v, lbl_v, hist_v, sem_h):
        @pl.when(j > 0)
        def _():
            pltpu.make_async_copy(hist_v, hist_hbm.at[base + i - 2], sem_h
                                  ).wait()
        for t in range(_CP // _L):
            hist_v[pl.ds(_L * t, _L)] = zero

        fs = [feat_all[i, pl.ds(_L * s, _L)] for s in range(_NS16)]

        @plsc.parallel_loop(0, _K, unroll=2)
        def _per_neighbour(k):
            acc = [zero, zero, zero, zero]
            for s in range(_NS16):
                diff = fs[s] - rows_v[k, pl.ds(_L * s, _L)]
                acc[s % 4] = acc[s % 4] + diff * diff
            pacc_v[pl.ds(k * _PS, _L)] = (acc[0] + acc[1]) + (acc[2] + acc[3])

        for g in range(_NG):
            part = [None] * 4
            for s in range(_NS16):
                col = plsc.load_gather(pacc_v, [k17[g] + s])
                part[s % 4] = col if part[s % 4] is None else part[s % 4] + col
            dist = (part[0] + part[1]) + (part[2] + part[3])
            ev = jnp.exp(w_v[pl.ds(_L * g, _L)] - _GC * dist)
            plsc.addupdate_scatter(hist_v, [lbl_v[pl.ds(_L * g, _L)]], ev)

        pltpu.make_async_copy(hist_v, hist_hbm.at[base + i], sem_h).start()

    pltpu.sync_copy(neigh_hbm.at[pl.ds(base, _BPW)], idx_all)
    pltpu.sync_copy(feat_hbm.at[pl.ds(base, _BPW)], feat_all)
    issue(0, rows_a, w_a, lbl_a, sem_a)

    @pl.loop(0, _BPW // 2)
    def _pair(j):
        a = 2 * j
        issue(a + 1, rows_b, w_b, lbl_b, sem_b)
        wait(a, rows_a, w_a, lbl_a, sem_a)
        compute(a, j, rows_a, w_a, lbl_a, hist_a, sem_ha)

        @pl.when(j < _BPW // 2 - 1)
        def _():
            issue(a + 2, rows_a, w_a, lbl_a, sem_a)

        wait(a + 1, rows_b, w_b, lbl_b, sem_b)
        compute(a + 1, j, rows_b, w_b, lbl_b, hist_b, sem_hb)

    pltpu.make_async_copy(hist_a, hist_hbm.at[base + _BPW - 2], sem_ha).wait()
    pltpu.make_async_copy(hist_b, hist_hbm.at[base + _BPW - 1], sem_hb).wait()


def _sc_class_sums(features, centres, centre_labels, neighbours, weight):
    mesh = plsc.VectorSubcoreMesh(core_axis_name="c", subcore_axis_name="s")
    cp = pltpu.CompilerParams()
    if "needs_layout_passes" in pltpu.CompilerParams.__dataclass_fields__:
        cp = dataclasses.replace(cp, needs_layout_passes=False)
    run = functools.partial(
        pl.kernel,
        out_type=jax.ShapeDtypeStruct((_B, _CP), jnp.float32),
        mesh=mesh,
        compiler_params=cp,
        scratch_types=[
            pltpu.VMEM((_BPW, _K), jnp.int32),
            pltpu.VMEM((_BPW, _D), jnp.float32),
            pltpu.VMEM((_K, _D), jnp.float32),
            pltpu.VMEM((_K, _D), jnp.float32),
            pltpu.VMEM((_K,), jnp.float32),
            pltpu.VMEM((_K,), jnp.float32),
            pltpu.VMEM((_K,), jnp.int32),
            pltpu.VMEM((_K,), jnp.int32),
            pltpu.VMEM((_K * _PS,), jnp.float32),
            pltpu.VMEM((_CP,), jnp.float32),
            pltpu.VMEM((_CP,), jnp.float32),
            pltpu.SemaphoreType.DMA,
            pltpu.SemaphoreType.DMA,
            pltpu.SemaphoreType.DMA,
            pltpu.SemaphoreType.DMA,
        ],
    )(_sc_body)
    return run(features, centres, centre_labels, neighbours, weight)


def _finalize_body(h_ref, o_ref):
    p = h_ref[...]
    p = jnp.where(p == 0.0, 1e-10, p)
    real = jax.lax.broadcasted_iota(jnp.int32, p.shape, 1) < _C
    s = jnp.sum(jnp.where(real, p, 0.0), axis=1, keepdims=True)
    logp = jnp.log(p / s)
    o_ref[...] = logp[:, :_C]


_finalize = pl.pallas_call(
    _finalize_body,
    grid=(2,),
    in_specs=[pl.BlockSpec((_B // 2, _CP), lambda i: (i, 0))],
    out_specs=pl.BlockSpec((_B // 2, _C), lambda i: (i, 0)),
    out_shape=jax.ShapeDtypeStruct((_B, _C), jnp.float32),
)


def kernel(features, centres, centre_labels, neighbours, weight):
    hist = _sc_class_sums(
        features,
        centres,
        centre_labels.astype(jnp.int32),
        neighbours.astype(jnp.int32),
        weight,
    )
    return _finalize(hist)
